# baseline (speedup 1.0000x reference)
import dataclasses

import jax
import jax.numpy as jnp
from jax import lax
from jax.experimental import pallas as pl
from jax.experimental.pallas import tpu as pltpu
from jax.experimental.pallas import tpu_sc as plsc

D = 64
L = 16
BT = 128
ST = 8
SS = 2
R = SS * BT
NG = 4
SCALE = 8.0

_cp = pltpu.CompilerParams(use_tc_tiling_on_sc=False)
if "needs_layout_passes" in pltpu.CompilerParams.__dataclass_fields__:
    _cp = dataclasses.replace(_cp, needs_layout_passes=False)


def kernel(x, table):
    b, s = x.shape
    nb, ns = b // BT, s // ST
    n_tiles = 32
    pairs_per_tile = (ns * nb) // n_tiles
    items_per_tile = pairs_per_tile * (ST // SS)
    x5 = (x.reshape(nb, BT, ns, ST).transpose(2, 0, 3, 1)
          .reshape(ns * nb, ST * BT))

    mesh = plsc.VectorSubcoreMesh(core_axis_name="core",
                                  subcore_axis_name="subcore")

    @pl.kernel(out_type=jax.ShapeDtypeStruct((s, D // ST, nb, ST * BT),
                                             jnp.float32),
               mesh=mesh,
               scratch_types=(
                   [pltpu.VMEM((pairs_per_tile, ST * BT), jnp.int32)]
                   + [pltpu.VMEM((R, D), jnp.float32)] * NG
                   + [pltpu.VMEM((SS, D // ST, ST * BT), jnp.float32)] * 2
                   + [pltpu.SemaphoreType.DMA] * (NG + 2)
               ),
               compiler_params=_cp)
    def run(table_hbm, x5_hbm, o5_hbm, idx_v, *bufs):
        gbuf = bufs[:NG]
        tbuf = bufs[NG:NG + 2]
        gsem = bufs[NG + 2:2 * NG + 2]
        osem = bufs[2 * NG + 2:2 * NG + 4]
        wid = lax.axis_index("subcore") * 2 + lax.axis_index("core")
        n_items = items_per_tile

        row_ids = [jnp.arange(l * L, (l + 1) * L, dtype=jnp.int32)
                   for l in range(R // L)]

        def gather_src(k):
            return table_hbm.at[idx_v.at[k // (ST // SS),
                                         pl.ds((k % (ST // SS)) * R, R)]]

        def out_dst(k):
            p = wid * pairs_per_tile + k // (ST // SS)
            s_out = (p // nb) * ST + (k % (ST // SS)) * SS
            bt = p % nb
            return o5_hbm.at[pl.ds(s_out, SS), :, bt]

        pltpu.sync_copy(x5_hbm.at[pl.ds(wid * pairs_per_tile,
                                        pairs_per_tile)],
                        idx_v)

        for k0 in range(NG - 1):
            pltpu.make_async_copy(gather_src(k0), gbuf[k0], gsem[k0]).start()

        @pl.loop(0, n_items // NG)
        def _(j):
            for u in range(NG):
                k = NG * j + u
                g, t = gbuf[u], tbuf[u % 2]

                @pl.when(k + NG - 1 < n_items)
                def _():
                    pltpu.make_async_copy(gather_src(k + NG - 1),
                                          gbuf[(u + NG - 1) % NG],
                                          gsem[(u + NG - 1) % NG]).start()

                pltpu.make_async_copy(gather_src(k), g, gsem[u]).wait()

                @pl.when(k >= 2)
                def _():
                    pltpu.make_async_copy(t, out_dst(k), osem[u % 2]).wait()

                iv_row = idx_v.at[k // (ST // SS)]
                fvs = [jnp.where(
                    iv_row[pl.ds((k % (ST // SS)) * R + l * L, L)] != 0,
                    SCALE, 0.0).astype(jnp.float32)
                       for l in range(R // L)]

                @plsc.parallel_loop(0, D, unroll=8)
                def _(d):
                    col = jnp.full((L,), d, jnp.int32)
                    base = (d % ST) * BT
                    for l in range(R // L):
                        v = plsc.load_gather(g, [row_ids[l], col])
                        dst = t.at[l // (BT // L), d // ST]
                        lb = l % (BT // L)
                        dst[pl.ds(base + lb * L, L)] = v * fvs[l]

                pltpu.make_async_copy(t, out_dst(k), osem[u % 2]).start()

        for k in (n_items - 2, n_items - 1):
            pltpu.make_async_copy(tbuf[k % 2], out_dst(k),
                                  osem[k % 2]).wait()

    out5 = run(table, x5)
    out = (out5.reshape(s, D // ST, nb, ST, BT)
           .transpose(2, 4, 0, 1, 3).reshape(b, s, D))
    return out


# speedup vs baseline: 1.0185x; 1.0185x over previous
import dataclasses

import jax
import jax.numpy as jnp
from jax import lax
from jax.experimental import pallas as pl
from jax.experimental.pallas import tpu as pltpu
from jax.experimental.pallas import tpu_sc as plsc

D = 64
L = 16
BT = 128
ST = 8
SS = 2
R = SS * BT
NG = 4
SCALE = 8.0

_cp = pltpu.CompilerParams(use_tc_tiling_on_sc=False)
if "needs_layout_passes" in pltpu.CompilerParams.__dataclass_fields__:
    _cp = dataclasses.replace(_cp, needs_layout_passes=False)


def kernel(x, table):
    b, s = x.shape
    nb, ns = b // BT, s // ST
    n_tiles = 32
    pairs_per_tile = (ns * nb) // n_tiles
    items_per_tile = pairs_per_tile * (ST // SS)
    x5 = (x.reshape(nb, BT, ns, ST).transpose(2, 0, 3, 1)
          .reshape(ns * nb, ST * BT))

    mesh = plsc.VectorSubcoreMesh(core_axis_name="core",
                                  subcore_axis_name="subcore")

    @pl.kernel(out_type=jax.ShapeDtypeStruct((s, D // ST, nb, ST * BT),
                                             jnp.float32),
               mesh=mesh,
               scratch_types=(
                   [pltpu.VMEM((pairs_per_tile, ST * BT), jnp.int32)]
                   + [pltpu.VMEM((R, D), jnp.float32)] * NG
                   + [pltpu.VMEM((SS, D // ST, ST * BT), jnp.float32)] * 2
                   + [pltpu.SemaphoreType.DMA] * (NG + 2)
               ),
               compiler_params=_cp)
    def run(table_hbm, x5_hbm, o5_hbm, idx_v, *bufs):
        gbuf = bufs[:NG]
        tbuf = bufs[NG:NG + 2]
        gsem = bufs[NG + 2:2 * NG + 2]
        osem = bufs[2 * NG + 2:2 * NG + 4]
        wid = lax.axis_index("subcore") * 2 + lax.axis_index("core")
        n_items = items_per_tile

        row_ids = [jnp.arange(l * L, (l + 1) * L, dtype=jnp.int32)
                   for l in range(R // L)]

        def gather_src(k):
            return table_hbm.at[idx_v.at[k // (ST // SS),
                                         pl.ds((k % (ST // SS)) * R, R)]]

        def out_dst(k):
            p = wid * pairs_per_tile + k // (ST // SS)
            s_out = (p // nb) * ST + (k % (ST // SS)) * SS
            bt = p % nb
            return o5_hbm.at[pl.ds(s_out, SS), :, bt]

        pltpu.sync_copy(x5_hbm.at[pl.ds(wid * pairs_per_tile,
                                        pairs_per_tile)],
                        idx_v)

        for k0 in range(NG - 1):
            pltpu.make_async_copy(gather_src(k0), gbuf[k0], gsem[k0]).start()

        @pl.loop(0, n_items // NG)
        def _(j):
            for u in range(NG):
                k = NG * j + u
                g, t = gbuf[u], tbuf[u % 2]

                @pl.when(k + NG - 1 < n_items)
                def _():
                    pltpu.make_async_copy(gather_src(k + NG - 1),
                                          gbuf[(u + NG - 1) % NG],
                                          gsem[(u + NG - 1) % NG]).start()

                pltpu.make_async_copy(gather_src(k), g, gsem[u]).wait()

                @pl.when(k >= 2)
                def _():
                    pltpu.make_async_copy(t, out_dst(k), osem[u % 2]).wait()

                iv_row = idx_v.at[k // (ST // SS)]
                fvs = [jnp.where(
                    iv_row[pl.ds((k % (ST // SS)) * R + l * L, L)] != 0,
                    SCALE, 0.0).astype(jnp.float32)
                       for l in range(R // L)]

                @plsc.parallel_loop(0, D, unroll=4)
                def _(d):
                    col = jnp.full((L,), d, jnp.int32)
                    base = (d % ST) * BT
                    for l in range(R // L):
                        v = plsc.load_gather(g, [row_ids[l], col])
                        dst = t.at[l // (BT // L), d // ST]
                        lb = l % (BT // L)
                        dst[pl.ds(base + lb * L, L)] = v * fvs[l]

                pltpu.make_async_copy(t, out_dst(k), osem[u % 2]).start()

        for k in (n_items - 2, n_items - 1):
            pltpu.make_async_copy(tbuf[k % 2], out_dst(k),
                                  osem[k % 2]).wait()

    out5 = run(table, x5)
    out = (out5.reshape(s, D // ST, nb, ST, BT)
           .transpose(2, 4, 0, 1, 3).reshape(b, s, D))
    return out
